# baseline (speedup 1.0000x reference)
import functools

import jax
import jax.numpy as jnp
from jax import lax
from jax.experimental import pallas as pl
from jax.experimental.pallas import tpu as pltpu
from jax.experimental.pallas import tpu_sc as plsc

NUM_TOKENS = 16384
INPUT_DIM = 2048
NUM_EXPERTS = 64
TOPK = 2

NUM_WORKERS = 32
SLAB = 512
LANES = 16

CHUNKS = 4
NT = NUM_TOKENS // CHUNKS

TC_BT = 2048
TC_PER = TC_BT // SLAB


def _tc_logits_body(x_ref, w_ref, out_ref):
    res = lax.dot_general(
        w_ref[...], x_ref[...],
        dimension_numbers=(((1,), (1,)), ((), ())),
        preferred_element_type=jnp.float32,
    )
    for j in range(TC_PER):
        out_ref[j, :, :] = res[:, j * SLAB:(j + 1) * SLAB]


def _tc_logits(x, W, ci):
    steps = NT // TC_BT
    return pl.pallas_call(
        _tc_logits_body,
        grid=(steps,),
        in_specs=[
            pl.BlockSpec((TC_BT, INPUT_DIM),
                         lambda i, ci=ci, steps=steps: (ci * steps + i, 0)),
            pl.BlockSpec((NUM_EXPERTS, INPUT_DIM), lambda i: (0, 0)),
        ],
        out_specs=pl.BlockSpec(
            (TC_PER, NUM_EXPERTS, SLAB), lambda i: (i, 0, 0)),
        out_shape=jax.ShapeDtypeStruct(
            (NT // SLAB, NUM_EXPERTS, SLAB), jnp.float32),
    )(x, W)


BLKW = NT // NUM_WORKERS
PER_SLAB = SLAB // BLKW


def _sc_route_body(lt_hbm, fw_hbm, ix_hbm, lt_v, fw_v, ix_v, sem):
    del sem
    c = lax.axis_index("c")
    s = lax.axis_index("s")
    wid = s * 2 + c
    base = wid * BLKW
    if PER_SLAB == 1:
        pltpu.sync_copy(lt_hbm.at[wid], lt_v)
    else:
        slab = wid // PER_SLAB
        col0 = (wid % PER_SLAB) * BLKW
        pltpu.sync_copy(lt_hbm.at[slab, :, pl.ds(col0, BLKW)], lt_v)

    lane = lax.iota(jnp.int32, LANES)

    def group(g, carry):
        t0 = g * LANES
        for e in range(NUM_EXPERTS):
            fw_v[e, pl.ds(t0, LANES)] = jnp.zeros((LANES,), jnp.float32)
        m1 = lt_v[0, pl.ds(t0, LANES)]
        i1 = jnp.zeros((LANES,), jnp.int32)
        m2 = jnp.full((LANES,), -jnp.inf, jnp.float32)
        i2 = jnp.zeros((LANES,), jnp.int32)
        for e in range(1, NUM_EXPERTS):
            v = lt_v[e, pl.ds(t0, LANES)]
            ev = jnp.full((LANES,), e, jnp.int32)
            gt1 = v > m1
            gt2 = v > m2
            m2 = jnp.where(gt2, jnp.where(gt1, m1, v), m2)
            i2 = jnp.where(gt2, jnp.where(gt1, i1, ev), i2)
            m1 = jnp.where(gt1, v, m1)
            i1 = jnp.where(gt1, ev, i1)
        ed = jnp.exp(m2 - m1)
        denom = 1.0 + ed
        w1 = 1.0 / denom
        w2 = ed / denom
        tloc = t0 + lane
        plsc.store_scatter(fw_v, [i1, tloc], w1)
        plsc.store_scatter(fw_v, [i2, tloc], w2)
        ix_v[0, pl.ds(t0, LANES)] = i1
        ix_v[1, pl.ds(t0, LANES)] = i2
        return carry

    lax.fori_loop(0, BLKW // LANES, group, 0)

    pltpu.sync_copy(fw_v, fw_hbm.at[:, pl.ds(base, BLKW)])
    pltpu.sync_copy(ix_v, ix_hbm.at[:, pl.ds(base, BLKW)])


@functools.cache
def _sc_route():
    return pl.kernel(
        _sc_route_body,
        out_type=(
            jax.ShapeDtypeStruct((NUM_EXPERTS, NT), jnp.float32),
            jax.ShapeDtypeStruct((TOPK, NT), jnp.int32),
        ),
        mesh=plsc.VectorSubcoreMesh(
            core_axis_name="c", subcore_axis_name="s",
            num_cores=2, num_subcores=16),
        scratch_types=[
            pltpu.VMEM((NUM_EXPERTS, BLKW), jnp.float32),
            pltpu.VMEM((NUM_EXPERTS, BLKW), jnp.float32),
            pltpu.VMEM((TOPK, BLKW), jnp.int32),
            pltpu.SemaphoreType.DMA,
        ],
        compiler_params=pltpu.CompilerParams(needs_layout_passes=False),
    )


def kernel(x, W):
    fws, ixs = [], []
    for ci in range(CHUNKS):
        lt = _tc_logits(x, W, ci)
        fw_em, ix_em = _sc_route()(lt)
        fws.append(fw_em)
        ixs.append(ix_em)
    if CHUNKS > 1:
        fw_em = jnp.concatenate(fws, axis=1)
        ix_em = jnp.concatenate(ixs, axis=1)
    else:
        fw_em, ix_em = fws[0], ixs[0]
    return (fw_em.T, ix_em.T)


# speedup vs baseline: 1.1860x; 1.1860x over previous
import functools

import jax
import jax.numpy as jnp
from jax import lax
from jax.experimental import pallas as pl
from jax.experimental.pallas import tpu as pltpu
from jax.experimental.pallas import tpu_sc as plsc

NUM_TOKENS = 16384
INPUT_DIM = 2048
NUM_EXPERTS = 64
TOPK = 2

NUM_WORKERS = 32
BLK = NUM_TOKENS // NUM_WORKERS
LANES = 16
GROUPS = BLK // LANES


TC_BT = 2048
TC_PER = TC_BT // BLK


def _tc_logits_body(x_ref, w_ref, out_ref):
    res = lax.dot_general(
        w_ref[...], x_ref[...],
        dimension_numbers=(((1,), (1,)), ((), ())),
        preferred_element_type=jnp.float32,
    )
    for j in range(TC_PER):
        out_ref[j, :, :] = res[:, j * BLK:(j + 1) * BLK]


def _tc_logits(x, W):
    return pl.pallas_call(
        _tc_logits_body,
        grid=(NUM_TOKENS // TC_BT,),
        in_specs=[
            pl.BlockSpec((TC_BT, INPUT_DIM), lambda i: (i, 0)),
            pl.BlockSpec((NUM_EXPERTS, INPUT_DIM), lambda i: (0, 0)),
        ],
        out_specs=pl.BlockSpec(
            (TC_PER, NUM_EXPERTS, BLK), lambda i: (i, 0, 0)),
        out_shape=jax.ShapeDtypeStruct(
            (NUM_WORKERS, NUM_EXPERTS, BLK), jnp.float32),
    )(x, W)


def _sc_route_body(lt_hbm, fw_hbm, ix_hbm, lt_v, fw_v, ix_v, sem):
    c = lax.axis_index("c")
    s = lax.axis_index("s")
    wid = s * 2 + c
    base = wid * BLK
    cp = pltpu.make_async_copy(lt_hbm.at[wid], lt_v, sem)
    cp.start()
    for e in range(NUM_EXPERTS):
        for q in range(GROUPS):
            fw_v[e, pl.ds(q * LANES, LANES)] = jnp.zeros(
                (LANES,), jnp.float32)
    cp.wait()

    lane = lax.iota(jnp.int32, LANES)

    def group(g, carry):
        t0 = g * LANES
        m1 = lt_v[0, pl.ds(t0, LANES)]
        i1 = jnp.zeros((LANES,), jnp.int32)
        m2 = jnp.full((LANES,), -jnp.inf, jnp.float32)
        i2 = jnp.zeros((LANES,), jnp.int32)
        for e in range(1, NUM_EXPERTS):
            v = lt_v[e, pl.ds(t0, LANES)]
            ev = jnp.full((LANES,), e, jnp.int32)
            gt1 = v > m1
            gt2 = v > m2
            m2 = jnp.where(gt2, jnp.where(gt1, m1, v), m2)
            i2 = jnp.where(gt2, jnp.where(gt1, i1, ev), i2)
            m1 = jnp.where(gt1, v, m1)
            i1 = jnp.where(gt1, ev, i1)
        ed = jnp.exp(m2 - m1)
        denom = 1.0 + ed
        w1 = 1.0 / denom
        w2 = ed / denom
        tloc = t0 + lane
        plsc.store_scatter(fw_v, [i1, tloc], w1)
        plsc.store_scatter(fw_v, [i2, tloc], w2)
        ix_v[0, pl.ds(t0, LANES)] = i1
        ix_v[1, pl.ds(t0, LANES)] = i2
        return carry

    lax.fori_loop(0, GROUPS, group, 0)

    pltpu.sync_copy(fw_v, fw_hbm.at[:, pl.ds(base, BLK)])
    pltpu.sync_copy(ix_v, ix_hbm.at[:, pl.ds(base, BLK)])


@functools.cache
def _sc_route():
    return pl.kernel(
        _sc_route_body,
        out_type=(
            jax.ShapeDtypeStruct((NUM_EXPERTS, NUM_TOKENS), jnp.float32),
            jax.ShapeDtypeStruct((TOPK, NUM_TOKENS), jnp.int32),
        ),
        mesh=plsc.VectorSubcoreMesh(
            core_axis_name="c", subcore_axis_name="s",
            num_cores=2, num_subcores=16),
        scratch_types=[
            pltpu.VMEM((NUM_EXPERTS, BLK), jnp.float32),
            pltpu.VMEM((NUM_EXPERTS, BLK), jnp.float32),
            pltpu.VMEM((TOPK, BLK), jnp.int32),
            pltpu.SemaphoreType.DMA,
        ],
        compiler_params=pltpu.CompilerParams(needs_layout_passes=False),
    )


def kernel(x, W):
    lt = _tc_logits(x, W)
    fw_em, ix_em = _sc_route()(lt)
    return (fw_em.T, ix_em.T)


# speedup vs baseline: 1.2041x; 1.0153x over previous
import functools

import jax
import jax.numpy as jnp
from jax import lax
from jax.experimental import pallas as pl
from jax.experimental.pallas import tpu as pltpu
from jax.experimental.pallas import tpu_sc as plsc

NUM_TOKENS = 16384
INPUT_DIM = 2048
NUM_EXPERTS = 64
TOPK = 2

NUM_WORKERS = 32
BLK = NUM_TOKENS // NUM_WORKERS
LANES = 16
GROUPS = BLK // LANES


TC_BT = 2048
TC_PER = TC_BT // BLK


def _tc_logits_body(x_ref, w_ref, out_ref):
    res = lax.dot_general(
        w_ref[...], x_ref[...],
        dimension_numbers=(((1,), (1,)), ((), ())),
        preferred_element_type=jnp.float32,
    )
    for j in range(TC_PER):
        out_ref[j, :, :] = res[:, j * BLK:(j + 1) * BLK]


def _tc_logits(x, W):
    return pl.pallas_call(
        _tc_logits_body,
        grid=(NUM_TOKENS // TC_BT,),
        in_specs=[
            pl.BlockSpec((TC_BT, INPUT_DIM), lambda i: (i, 0)),
            pl.BlockSpec((NUM_EXPERTS, INPUT_DIM), lambda i: (0, 0)),
        ],
        out_specs=pl.BlockSpec(
            (TC_PER, NUM_EXPERTS, BLK), lambda i: (i, 0, 0)),
        out_shape=jax.ShapeDtypeStruct(
            (NUM_WORKERS, NUM_EXPERTS, BLK), jnp.float32),
    )(x, W)


def _sc_route_body(lt_hbm, fw_hbm, ix_hbm, lt_v, fw_v, ix_v, sem):
    del sem
    c = lax.axis_index("c")
    s = lax.axis_index("s")
    wid = s * 2 + c
    base = wid * BLK
    pltpu.sync_copy(lt_hbm.at[wid], lt_v)

    lane = lax.iota(jnp.int32, LANES)

    def group(g, carry):
        t0 = g * LANES
        for e in range(NUM_EXPERTS):
            fw_v[e, pl.ds(t0, LANES)] = jnp.zeros((LANES,), jnp.float32)
        m1 = lt_v[0, pl.ds(t0, LANES)]
        i1 = jnp.zeros((LANES,), jnp.int32)
        m2 = jnp.full((LANES,), -jnp.inf, jnp.float32)
        i2 = jnp.zeros((LANES,), jnp.int32)
        for e in range(1, NUM_EXPERTS):
            v = lt_v[e, pl.ds(t0, LANES)]
            ev = jnp.full((LANES,), e, jnp.int32)
            gt1 = v > m1
            gt2 = v > m2
            m2 = jnp.where(gt2, jnp.where(gt1, m1, v), m2)
            i2 = jnp.where(gt2, jnp.where(gt1, i1, ev), i2)
            m1 = jnp.where(gt1, v, m1)
            i1 = jnp.where(gt1, ev, i1)
        ed = jnp.exp(m2 - m1)
        denom = 1.0 + ed
        w1 = 1.0 / denom
        w2 = ed / denom
        tloc = t0 + lane
        plsc.store_scatter(fw_v, [i1, tloc], w1)
        plsc.store_scatter(fw_v, [i2, tloc], w2)
        ix_v[0, pl.ds(t0, LANES)] = i1
        ix_v[1, pl.ds(t0, LANES)] = i2
        return carry

    lax.fori_loop(0, GROUPS, group, 0)

    pltpu.sync_copy(fw_v, fw_hbm.at[:, pl.ds(base, BLK)])
    pltpu.sync_copy(ix_v, ix_hbm.at[:, pl.ds(base, BLK)])


@functools.cache
def _sc_route():
    return pl.kernel(
        _sc_route_body,
        out_type=(
            jax.ShapeDtypeStruct((NUM_EXPERTS, NUM_TOKENS), jnp.float32),
            jax.ShapeDtypeStruct((TOPK, NUM_TOKENS), jnp.int32),
        ),
        mesh=plsc.VectorSubcoreMesh(
            core_axis_name="c", subcore_axis_name="s",
            num_cores=2, num_subcores=16),
        scratch_types=[
            pltpu.VMEM((NUM_EXPERTS, BLK), jnp.float32),
            pltpu.VMEM((NUM_EXPERTS, BLK), jnp.float32),
            pltpu.VMEM((TOPK, BLK), jnp.int32),
            pltpu.SemaphoreType.DMA,
        ],
        compiler_params=pltpu.CompilerParams(needs_layout_passes=False),
    )


def kernel(x, W):
    lt = _tc_logits(x, W)
    fw_em, ix_em = _sc_route()(lt)
    return (fw_em.T, ix_em.T)


# speedup vs baseline: 1.2222x; 1.0150x over previous
import functools

import jax
import jax.numpy as jnp
from jax import lax
from jax.experimental import pallas as pl
from jax.experimental.pallas import tpu as pltpu
from jax.experimental.pallas import tpu_sc as plsc

NUM_TOKENS = 16384
INPUT_DIM = 2048
NUM_EXPERTS = 64
TOPK = 2

NUM_WORKERS = 32
BLK = NUM_TOKENS // NUM_WORKERS
LANES = 16
GROUPS = BLK // LANES


TC_BT = 1024
TC_PER = TC_BT // BLK


def _tc_logits_body(x_ref, w_ref, out_ref):
    res = lax.dot_general(
        w_ref[...], x_ref[...],
        dimension_numbers=(((1,), (1,)), ((), ())),
        preferred_element_type=jnp.float32,
    )
    for j in range(TC_PER):
        out_ref[j, :, :] = res[:, j * BLK:(j + 1) * BLK]


def _tc_logits(x, W):
    return pl.pallas_call(
        _tc_logits_body,
        grid=(NUM_TOKENS // TC_BT,),
        in_specs=[
            pl.BlockSpec((TC_BT, INPUT_DIM), lambda i: (i, 0)),
            pl.BlockSpec((NUM_EXPERTS, INPUT_DIM), lambda i: (0, 0)),
        ],
        out_specs=pl.BlockSpec(
            (TC_PER, NUM_EXPERTS, BLK), lambda i: (i, 0, 0)),
        out_shape=jax.ShapeDtypeStruct(
            (NUM_WORKERS, NUM_EXPERTS, BLK), jnp.float32),
    )(x, W)


def _sc_route_body(lt_hbm, fw_hbm, ix_hbm, lt_v, fw_v, ix_v, sem):
    del sem
    c = lax.axis_index("c")
    s = lax.axis_index("s")
    wid = s * 2 + c
    base = wid * BLK
    pltpu.sync_copy(lt_hbm.at[wid], lt_v)

    lane = lax.iota(jnp.int32, LANES)

    def group(g, carry):
        t0 = g * LANES
        for e in range(NUM_EXPERTS):
            fw_v[e, pl.ds(t0, LANES)] = jnp.zeros((LANES,), jnp.float32)
        m1 = lt_v[0, pl.ds(t0, LANES)]
        i1 = jnp.zeros((LANES,), jnp.int32)
        m2 = jnp.full((LANES,), -jnp.inf, jnp.float32)
        i2 = jnp.zeros((LANES,), jnp.int32)
        for e in range(1, NUM_EXPERTS):
            v = lt_v[e, pl.ds(t0, LANES)]
            ev = jnp.full((LANES,), e, jnp.int32)
            gt1 = v > m1
            gt2 = v > m2
            m2 = jnp.where(gt2, jnp.where(gt1, m1, v), m2)
            i2 = jnp.where(gt2, jnp.where(gt1, i1, ev), i2)
            m1 = jnp.where(gt1, v, m1)
            i1 = jnp.where(gt1, ev, i1)
        ed = jnp.exp(m2 - m1)
        denom = 1.0 + ed
        w1 = 1.0 / denom
        w2 = ed / denom
        tloc = t0 + lane
        plsc.store_scatter(fw_v, [i1, tloc], w1)
        plsc.store_scatter(fw_v, [i2, tloc], w2)
        ix_v[0, pl.ds(t0, LANES)] = i1
        ix_v[1, pl.ds(t0, LANES)] = i2
        return carry

    lax.fori_loop(0, GROUPS, group, 0)

    pltpu.sync_copy(fw_v, fw_hbm.at[:, pl.ds(base, BLK)])
    pltpu.sync_copy(ix_v, ix_hbm.at[:, pl.ds(base, BLK)])


@functools.cache
def _sc_route():
    return pl.kernel(
        _sc_route_body,
        out_type=(
            jax.ShapeDtypeStruct((NUM_EXPERTS, NUM_TOKENS), jnp.float32),
            jax.ShapeDtypeStruct((TOPK, NUM_TOKENS), jnp.int32),
        ),
        mesh=plsc.VectorSubcoreMesh(
            core_axis_name="c", subcore_axis_name="s",
            num_cores=2, num_subcores=16),
        scratch_types=[
            pltpu.VMEM((NUM_EXPERTS, BLK), jnp.float32),
            pltpu.VMEM((NUM_EXPERTS, BLK), jnp.float32),
            pltpu.VMEM((TOPK, BLK), jnp.int32),
            pltpu.SemaphoreType.DMA,
        ],
        compiler_params=pltpu.CompilerParams(needs_layout_passes=False),
    )


def kernel(x, W):
    lt = _tc_logits(x, W)
    fw_em, ix_em = _sc_route()(lt)
    return (fw_em.T, ix_em.T)
